# baseline (speedup 1.0000x reference)
import jax
import jax.numpy as jnp
from jax.experimental import pallas as pl
from jax.experimental.pallas import tpu as pltpu

_NBLK = 8192


def _body(f_ref, d_ref, w1_ref, b1_ref, wc_ref, bc_ref, wr_ref, br_ref,
          sem_ref, ang_ref, dist_ref):
    f = f_ref[0]
    x = jnp.dot(w1_ref[...], f, preferred_element_type=jnp.float32)
    x = jnp.maximum(x + b1_ref[...], 0.0)
    sem = jnp.dot(wc_ref[...], x, preferred_element_type=jnp.float32) + bc_ref[...]
    reg = jnp.dot(wr_ref[...], x, preferred_element_type=jnp.float32) + br_ref[...]
    sem_ref[0] = jnp.transpose(sem, (1, 0))
    ang_ref[0] = reg[0:1]
    dist_ref[0] = d_ref[0] + jnp.transpose(reg[1:7], (1, 0))


def kernel(fused_feats, obj_scores, distance, W1, b1, gamma1, beta1, Wc, bc, Wr, br):
    B, C, N = fused_feats.shape
    NUM_CLS = Wc.shape[0]
    W1f = W1 * gamma1[:, None]
    b1f = (b1 * gamma1 + beta1)[:, None]
    nb = pl.cdiv(N, _NBLK)

    out_shapes = (
        jax.ShapeDtypeStruct((B, N, NUM_CLS), jnp.float32),
        jax.ShapeDtypeStruct((B, 1, N), jnp.float32),
        jax.ShapeDtypeStruct((B, N, 6), jnp.float32),
    )
    sem, ang, dist = pl.pallas_call(
        _body,
        grid=(B, nb),
        in_specs=[
            pl.BlockSpec((1, C, _NBLK), lambda b, n: (b, 0, n)),
            pl.BlockSpec((1, _NBLK, 6), lambda b, n: (b, n, 0)),
            pl.BlockSpec((C, C), lambda b, n: (0, 0)),
            pl.BlockSpec((C, 1), lambda b, n: (0, 0)),
            pl.BlockSpec((NUM_CLS, C), lambda b, n: (0, 0)),
            pl.BlockSpec((NUM_CLS, 1), lambda b, n: (0, 0)),
            pl.BlockSpec((7, C), lambda b, n: (0, 0)),
            pl.BlockSpec((7, 1), lambda b, n: (0, 0)),
        ],
        out_specs=(
            pl.BlockSpec((1, _NBLK, NUM_CLS), lambda b, n: (b, n, 0)),
            pl.BlockSpec((1, 1, _NBLK), lambda b, n: (b, 0, n)),
            pl.BlockSpec((1, _NBLK, 6), lambda b, n: (b, n, 0)),
        ),
        out_shape=out_shapes,
        compiler_params=pltpu.CompilerParams(dimension_semantics=("parallel", "parallel")),
    )(fused_feats, distance, W1f, b1f, Wc, bc[:, None], Wr, br[:, None])
    return (sem, ang.reshape(B, N), dist, obj_scores)


# speedup vs baseline: 1.3370x; 1.3370x over previous
import jax
import jax.numpy as jnp
from jax.experimental import pallas as pl
from jax.experimental.pallas import tpu as pltpu

_NBLK = 8192
_CHUNKS = 2


def _body(f_ref, d_ref, w1_ref, b1_ref, wc_ref, bc_ref, wr_ref, br_ref,
          sem_ref, ang_ref, dist_ref):
    f = f_ref[0]
    x = jnp.dot(w1_ref[...], f, preferred_element_type=jnp.float32)
    x = jnp.maximum(x + b1_ref[...], 0.0)
    sem_ref[0] = jnp.dot(wc_ref[...], x, preferred_element_type=jnp.float32) + bc_ref[...]
    reg = jnp.dot(wr_ref[...], x, preferred_element_type=jnp.float32) + br_ref[...]
    ang_ref[0] = reg[0:1]
    dist_ref[0] = d_ref[0] + reg[1:7]


def _run_chunk(feats, dist_cm, W1f, b1f, Wc, bc2, Wr, br2):
    B, C, N = feats.shape
    NUM_CLS = Wc.shape[0]
    nb = pl.cdiv(N, _NBLK)
    out_shapes = (
        jax.ShapeDtypeStruct((B, NUM_CLS, N), jnp.float32),
        jax.ShapeDtypeStruct((B, 1, N), jnp.float32),
        jax.ShapeDtypeStruct((B, 6, N), jnp.float32),
    )
    return pl.pallas_call(
        _body,
        grid=(B, nb),
        in_specs=[
            pl.BlockSpec((1, C, _NBLK), lambda b, n: (b, 0, n)),
            pl.BlockSpec((1, 6, _NBLK), lambda b, n: (b, 0, n)),
            pl.BlockSpec((C, C), lambda b, n: (0, 0)),
            pl.BlockSpec((C, 1), lambda b, n: (0, 0)),
            pl.BlockSpec((NUM_CLS, C), lambda b, n: (0, 0)),
            pl.BlockSpec((NUM_CLS, 1), lambda b, n: (0, 0)),
            pl.BlockSpec((7, C), lambda b, n: (0, 0)),
            pl.BlockSpec((7, 1), lambda b, n: (0, 0)),
        ],
        out_specs=(
            pl.BlockSpec((1, NUM_CLS, _NBLK), lambda b, n: (b, 0, n)),
            pl.BlockSpec((1, 1, _NBLK), lambda b, n: (b, 0, n)),
            pl.BlockSpec((1, 6, _NBLK), lambda b, n: (b, 0, n)),
        ),
        out_shape=out_shapes,
        compiler_params=pltpu.CompilerParams(dimension_semantics=("parallel", "parallel")),
    )(feats, dist_cm, W1f, b1f, Wc, bc2, Wr, br2)


def kernel(fused_feats, obj_scores, distance, W1, b1, gamma1, beta1, Wc, bc, Wr, br):
    B, C, N = fused_feats.shape
    W1f = W1 * gamma1[:, None]
    b1f = (b1 * gamma1 + beta1)[:, None]
    bc2 = bc[:, None]
    br2 = br[:, None]
    dist_cm = jnp.transpose(distance, (0, 2, 1))

    bs = B // _CHUNKS
    sems, angs, dists = [], [], []
    for c in range(_CHUNKS):
        s, a, dt = _run_chunk(fused_feats[c * bs:(c + 1) * bs],
                              dist_cm[c * bs:(c + 1) * bs],
                              W1f, b1f, Wc, bc2, Wr, br2)
        sems.append(jnp.transpose(s, (0, 2, 1)))
        angs.append(a.reshape(bs, N))
        dists.append(jnp.transpose(dt, (0, 2, 1)))
    return (jnp.concatenate(sems, axis=0),
            jnp.concatenate(angs, axis=0),
            jnp.concatenate(dists, axis=0),
            obj_scores)


# speedup vs baseline: 1.7746x; 1.3273x over previous
import jax
import jax.numpy as jnp
from jax.experimental import pallas as pl
from jax.experimental.pallas import tpu as pltpu

_NBLK = 8192


def _body(f_ref, d_ref, w1_ref, b1_ref, wc_ref, bc_ref, wr_ref, br_ref,
          sem_ref, ang_ref, dist_ref):
    f = f_ref[0]
    x = jnp.dot(w1_ref[...], f, preferred_element_type=jnp.float32)
    x = jnp.maximum(x + b1_ref[...], 0.0)
    sem = jnp.dot(wc_ref[...], x, preferred_element_type=jnp.float32) + bc_ref[...]
    sem_ref[0] = sem.astype(jnp.bfloat16)
    reg = jnp.dot(wr_ref[...], x, preferred_element_type=jnp.float32) + br_ref[...]
    ang_ref[0] = reg[0:1]
    dist_ref[0] = d_ref[0] + reg[1:7]


def kernel(fused_feats, obj_scores, distance, W1, b1, gamma1, beta1, Wc, bc, Wr, br):
    B, C, N = fused_feats.shape
    NUM_CLS = Wc.shape[0]
    W1f = W1 * gamma1[:, None]
    b1f = (b1 * gamma1 + beta1)[:, None]
    dist_cm = jnp.transpose(distance, (0, 2, 1))
    nb = pl.cdiv(N, _NBLK)

    out_shapes = (
        jax.ShapeDtypeStruct((B, NUM_CLS, N), jnp.bfloat16),
        jax.ShapeDtypeStruct((B, 1, N), jnp.float32),
        jax.ShapeDtypeStruct((B, 6, N), jnp.float32),
    )
    sem_cm, ang, dist_cm_out = pl.pallas_call(
        _body,
        grid=(B, nb),
        in_specs=[
            pl.BlockSpec((1, C, _NBLK), lambda b, n: (b, 0, n)),
            pl.BlockSpec((1, 6, _NBLK), lambda b, n: (b, 0, n)),
            pl.BlockSpec((C, C), lambda b, n: (0, 0)),
            pl.BlockSpec((C, 1), lambda b, n: (0, 0)),
            pl.BlockSpec((NUM_CLS, C), lambda b, n: (0, 0)),
            pl.BlockSpec((NUM_CLS, 1), lambda b, n: (0, 0)),
            pl.BlockSpec((7, C), lambda b, n: (0, 0)),
            pl.BlockSpec((7, 1), lambda b, n: (0, 0)),
        ],
        out_specs=(
            pl.BlockSpec((1, NUM_CLS, _NBLK), lambda b, n: (b, 0, n)),
            pl.BlockSpec((1, 1, _NBLK), lambda b, n: (b, 0, n)),
            pl.BlockSpec((1, 6, _NBLK), lambda b, n: (b, 0, n)),
        ),
        out_shape=out_shapes,
        compiler_params=pltpu.CompilerParams(dimension_semantics=("parallel", "parallel")),
    )(fused_feats, dist_cm, W1f, b1f, Wc, bc[:, None], Wr, br[:, None])
    sem = jnp.transpose(sem_cm, (0, 2, 1)).astype(jnp.float32)
    dist = jnp.transpose(dist_cm_out, (0, 2, 1))
    return (sem, ang.reshape(B, N), dist, obj_scores)
